# baseline (speedup 1.0000x reference)
import functools

import jax
import jax.numpy as jnp
from jax import lax
from jax.experimental import pallas as pl
from jax.experimental.pallas import tpu as pltpu
from jax.experimental.pallas import tpu_sc as plsc

D = 128
NC = 2
NS = 16
NW = NC * NS
CHUNK = 128
RPT = 640
ACC_ROWS = NS * RPT


NB = 2
G = 20


def _sc_scatter_add(feat, idx5, zeros):
    ngrp = idx5.shape[1]
    mesh = plsc.VectorSubcoreMesh(core_axis_name="c", subcore_axis_name="s")

    @functools.partial(
        pl.kernel,
        mesh=mesh,
        out_type=jax.ShapeDtypeStruct((NC, ACC_ROWS, D), jnp.float32),
        scratch_types=[
            *[pltpu.VMEM((G, 2, CHUNK), jnp.int32) for _ in range(2)],
            *[pltpu.VMEM((CHUNK, D), jnp.float32) for _ in range(NB)],
            pltpu.VMEM_SHARED((ACC_ROWS, D), jnp.float32),
            *[pltpu.SemaphoreType.DMA for _ in range(NB + 3)],
        ],
    )
    def k(feat_h, idx_h, zeros_h, out_h, ib0, ib1, *rest):
        ibufs = (ib0, ib1)
        bufs = rest[:NB]
        acc_s = rest[NB]
        gsem = rest[NB + 1: 2 * NB + 1]
        isem = rest[2 * NB + 1: 2 * NB + 3]
        c = lax.axis_index("c")
        s = lax.axis_index("s")
        wid = s * NC + c
        zcopy = pltpu.async_copy(zeros_h, acc_s.at[pl.ds(s * RPT, RPT)],
                                 isem[1])
        pltpu.sync_copy(idx_h.at[wid, 0], ib0)
        zcopy.wait()
        plsc.subcore_barrier()

        for grp in range(ngrp):
            ib = ibufs[grp % 2]
            if grp > 0:
                pltpu.make_async_copy(
                    idx_h.at[wid, grp], ib, isem[grp % 2]
                ).wait()
            if grp + 1 < ngrp:
                pltpu.async_copy(
                    idx_h.at[wid, grp + 1], ibufs[(grp + 1) % 2],
                    isem[(grp + 1) % 2],
                )
            for b in range(NB):
                pltpu.async_copy(feat_h.at[ib.at[b, 0]], bufs[b], gsem[b])

            def body(i, _):
                for b in range(NB):
                    t = i * NB + b
                    pltpu.make_async_copy(
                        feat_h.at[ib.at[t, 0]], bufs[b], gsem[b]
                    ).wait()
                    pltpu.sync_copy(bufs[b], acc_s.at[ib.at[t, 1]], add=True)
                    pltpu.async_copy(
                        feat_h.at[ib.at[t + NB, 0]], bufs[b], gsem[b]
                    )
                return ()

            lax.fori_loop(0, (G - NB) // NB, body, ())
            for b in range(NB):
                t = G - NB + b
                pltpu.make_async_copy(
                    feat_h.at[ib.at[t, 0]], bufs[b], gsem[b]
                ).wait()
                pltpu.sync_copy(bufs[b], acc_s.at[ib.at[t, 1]], add=True)

        plsc.subcore_barrier()
        pltpu.sync_copy(
            acc_s.at[pl.ds(s * RPT, RPT)], out_h.at[c, pl.ds(s * RPT, RPT)]
        )

    return k(feat, idx5, zeros)


def _tc_linear(partials, W, b, n_nodes):
    blk = 1000
    grid = n_nodes // blk

    def body(p_ref, w_ref, b_ref, o_ref):
        x = p_ref[0] + p_ref[1]
        y = lax.dot_general(
            x, w_ref[...], (((1,), (1,)), ((), ())),
            preferred_element_type=jnp.float32,
        )
        o_ref[...] = y + b_ref[...]

    return pl.pallas_call(
        body,
        grid=(grid,),
        in_specs=[
            pl.BlockSpec((NC, blk, D), lambda i: (0, i, 0)),
            pl.BlockSpec((D, D), lambda i: (0, 0)),
            pl.BlockSpec((1, D), lambda i: (0, 0)),
        ],
        out_specs=pl.BlockSpec((blk, D), lambda i: (i, 0)),
        out_shape=jax.ShapeDtypeStruct((n_nodes, D), jnp.float32),
    )(partials, W, b.reshape(1, D))


def kernel(feat, edge_index, W, b):
    n_nodes = feat.shape[0]
    n_edges = edge_index.shape[1]
    src = edge_index[0].astype(jnp.int32)
    dst = edge_index[1].astype(jnp.int32)
    epw = NW * CHUNK * G
    e_pad = ((n_edges + epw - 1) // epw) * epw
    pad = e_pad - n_edges
    if pad:
        src = jnp.concatenate([src, jnp.zeros((pad,), jnp.int32)])
        dst = jnp.concatenate([dst, jnp.full((pad,), ACC_ROWS - 1, jnp.int32)])
    cpt = e_pad // (NW * CHUNK)
    ngrp = cpt // G
    src3 = src.reshape(NW, cpt, CHUNK)
    dst3 = dst.reshape(NW, cpt, CHUNK)
    idx5 = jnp.stack([src3, dst3], axis=2).reshape(NW, ngrp, G, 2, CHUNK)
    zeros = jnp.zeros((RPT, D), jnp.float32)
    partials = _sc_scatter_add(feat, idx5, zeros)
    return _tc_linear(partials, W, b, n_nodes)


# speedup vs baseline: 1.2293x; 1.2293x over previous
import functools

import jax
import jax.numpy as jnp
from jax import lax
from jax.experimental import pallas as pl
from jax.experimental.pallas import tpu as pltpu
from jax.experimental.pallas import tpu_sc as plsc

D = 128
NC = 2
NS = 16
NW = NC * NS
CHUNK = 128
RPT = 640
ACC_ROWS = NS * RPT


NB = 2
G = 20


def _sc_scatter_add(feat, idx5, zeros):
    ngrp = idx5.shape[1]
    mesh = plsc.VectorSubcoreMesh(core_axis_name="c", subcore_axis_name="s")

    @functools.partial(
        pl.kernel,
        mesh=mesh,
        out_type=jax.ShapeDtypeStruct((NC, ACC_ROWS, D), jnp.float32),
        scratch_types=[
            *[pltpu.VMEM((G, 2, CHUNK), jnp.int32) for _ in range(2)],
            *[pltpu.VMEM((CHUNK, D), jnp.float32) for _ in range(NB)],
            pltpu.VMEM_SHARED((ACC_ROWS, D), jnp.float32),
            *[pltpu.SemaphoreType.DMA for _ in range(NB + 3)],
        ],
    )
    def k(feat_h, idx_h, zeros_h, out_h, ib0, ib1, *rest):
        ibufs = (ib0, ib1)
        bufs = rest[:NB]
        acc_s = rest[NB]
        gsem = rest[NB + 1: 2 * NB + 1]
        isem = rest[2 * NB + 1: 2 * NB + 3]
        c = lax.axis_index("c")
        s = lax.axis_index("s")
        wid = s * NC + c
        zcopy = pltpu.async_copy(zeros_h, acc_s.at[pl.ds(s * RPT, RPT)],
                                 isem[1])
        pltpu.sync_copy(idx_h.at[wid, 0], ib0)
        zcopy.wait()
        plsc.subcore_barrier()

        for grp in range(ngrp):
            ib = ibufs[grp % 2]
            if grp > 0:
                pltpu.make_async_copy(
                    idx_h.at[wid, grp], ib, isem[grp % 2]
                ).wait()
            if grp + 1 < ngrp:
                pltpu.async_copy(
                    idx_h.at[wid, grp + 1], ibufs[(grp + 1) % 2],
                    isem[(grp + 1) % 2],
                )
            for b in range(NB):
                pltpu.async_copy(feat_h.at[ib.at[b, 0]], bufs[b], gsem[b])

            def body(i, _):
                for b in range(NB):
                    t = i * NB + b
                    pltpu.make_async_copy(
                        feat_h.at[ib.at[t, 0]], bufs[b], gsem[b]
                    ).wait()
                    pltpu.sync_copy(bufs[b], acc_s.at[ib.at[t, 1]], add=True)
                    pltpu.async_copy(
                        feat_h.at[ib.at[t + NB, 0]], bufs[b], gsem[b]
                    )
                return ()

            lax.fori_loop(0, (G - NB) // NB, body, ())
            for b in range(NB):
                t = G - NB + b
                pltpu.make_async_copy(
                    feat_h.at[ib.at[t, 0]], bufs[b], gsem[b]
                ).wait()
                pltpu.sync_copy(bufs[b], acc_s.at[ib.at[t, 1]], add=True)

        plsc.subcore_barrier()
        pltpu.sync_copy(
            acc_s.at[pl.ds(s * RPT, RPT)], out_h.at[c, pl.ds(s * RPT, RPT)]
        )

    return k(feat, idx5, zeros)


def _tc_linear(partials, W, b, n_nodes):
    blk = 1000
    grid = n_nodes // blk

    def body(p_ref, w_ref, b_ref, o_ref):
        x = p_ref[0] + p_ref[1]
        y = lax.dot_general(
            x, w_ref[...], (((1,), (1,)), ((), ())),
            preferred_element_type=jnp.float32,
        )
        o_ref[...] = y + b_ref[...]

    return pl.pallas_call(
        body,
        grid=(grid,),
        in_specs=[
            pl.BlockSpec((NC, blk, D), lambda i: (0, i, 0)),
            pl.BlockSpec((D, D), lambda i: (0, 0)),
            pl.BlockSpec((1, D), lambda i: (0, 0)),
        ],
        out_specs=pl.BlockSpec((blk, D), lambda i: (i, 0)),
        out_shape=jax.ShapeDtypeStruct((n_nodes, D), jnp.float32),
    )(partials, W, b.reshape(1, D))


def kernel(feat, edge_index, W, b):
    n_nodes = feat.shape[0]
    n_edges = edge_index.shape[1]
    src = edge_index[0].astype(jnp.int32)
    dst = edge_index[1].astype(jnp.int32)
    epw = NW * CHUNK * G
    e_pad = ((n_edges + epw - 1) // epw) * epw
    pad = e_pad - n_edges
    if pad:
        src = jnp.concatenate([src, jnp.zeros((pad,), jnp.int32)])
        dst = jnp.concatenate([dst, jnp.full((pad,), ACC_ROWS - 1, jnp.int32)])
    cpt = e_pad // (NW * CHUNK)
    ngrp = cpt // G
    src3 = src.reshape(NW, cpt, CHUNK)
    core_of_wid = (jnp.arange(NW, dtype=jnp.int32) % NC)[:, None, None]
    src3 = src3 + core_of_wid * n_nodes
    dst3 = dst.reshape(NW, cpt, CHUNK)
    idx5 = jnp.stack([src3, dst3], axis=2).reshape(NW, ngrp, G, 2, CHUNK)
    zeros = jnp.zeros((RPT, D), jnp.float32)
    feat2 = jnp.concatenate([feat, feat], axis=0)
    partials = _sc_scatter_add(feat2, idx5, zeros)
    return _tc_linear(partials, W, b, n_nodes)
